# baseline (speedup 1.0000x reference)
import functools

import jax
import jax.numpy as jnp
from jax import lax
from jax.experimental import pallas as pl
from jax.experimental.pallas import tpu as pltpu
from jax.experimental.pallas import tpu_sc as plsc

N = 10000
E = 320000
D = 128

NC = 2
NS = 16
NW = NC * NS
CHUNK = 128
NCHUNK = 80
EPW = NCHUNK * CHUNK
EPAD = NW * EPW
QCH = 16
NP = 10240
ROWS_PER_TILE = NP // NS
ZROWS = 128
NSLOT = 2
LOOKAHEAD = 1


@functools.cache
def _make_sc_kernel(with_deg):
    out_type = [jax.ShapeDtypeStruct((NC * NP, D), jnp.float32)]
    scratch = [
        pltpu.VMEM_SHARED((NP, D), jnp.float32),
        pltpu.VMEM((EPW,), jnp.int32),
        pltpu.VMEM((QCH, CHUNK), jnp.int32),
        [pltpu.VMEM((CHUNK, D), jnp.float32) for _ in range(NSLOT)],
        [pltpu.SemaphoreType.DMA for _ in range(NSLOT)],
    ]
    if with_deg:
        out_type.append(jax.ShapeDtypeStruct((NC * NP,), jnp.float32))
        scratch += [
            pltpu.VMEM_SHARED((NP,), jnp.float32),
            pltpu.VMEM((CHUNK,), jnp.float32),
            pltpu.VMEM((ROWS_PER_TILE,), jnp.float32),
        ]

    mesh = plsc.VectorSubcoreMesh(core_axis_name="c", subcore_axis_name="s",
                                  num_cores=NC, num_subcores=NS)

    def body(*refs):
        if with_deg:
            (x_hbm, src_hbm, dst_hbm, sum_hbm, deg_hbm,
             acc, src_all, dst_all, rows, sems,
             dacc, ones_v, dzbuf) = refs
        else:
            (x_hbm, src_hbm, dst_hbm, sum_hbm,
             acc, src_all, dst_all, rows, sems) = refs
        zbuf = rows[0]
        c = lax.axis_index("c")
        s = lax.axis_index("s")
        wid = c * NS + s

        def zfill(i, carry):
            r = i // (D // 16)
            col = (i % (D // 16)) * 16
            zbuf[r, pl.ds(col, 16)] = jnp.zeros((16,), jnp.float32)
            return carry
        lax.fori_loop(0, CHUNK * (D // 16), zfill, 0)
        if with_deg:
            def ofill(i, carry):
                ones_v[pl.ds(i * 16, 16)] = jnp.ones((16,), jnp.float32)
                return carry
            lax.fori_loop(0, CHUNK // 16, ofill, 0)

            def dzfill(i, carry):
                dzbuf[pl.ds(i * 16, 16)] = jnp.zeros((16,), jnp.float32)
                return carry
            lax.fori_loop(0, ROWS_PER_TILE // 16, dzfill, 0)

        row0 = s * ROWS_PER_TILE

        def zcopy(i, carry):
            pltpu.sync_copy(zbuf, acc.at[pl.ds(row0 + i * CHUNK, CHUNK)])
            return carry
        lax.fori_loop(0, ROWS_PER_TILE // CHUNK, zcopy, 0)
        if with_deg:
            pltpu.sync_copy(dzbuf, dacc.at[pl.ds(row0, ROWS_PER_TILE)])
        plsc.subcore_barrier()

        pltpu.sync_copy(src_hbm.at[pl.ds(wid * EPW, EPW)], src_all)
        pltpu.sync_copy(dst_hbm.at[wid, pl.ds(0, QCH)], dst_all)

        dummy = x_hbm.at[pl.ds(0, CHUNK)]
        dummy_deg = x_hbm.at[0, pl.ds(0, CHUNK)]
        NGROUP = (NCHUNK + LOOKAHEAD + NSLOT - 1) // NSLOT

        def pipe_group(g, carry):
            for b in range(NSLOT):
                t = g * NSLOT + b

                @pl.when(t < NCHUNK)
                def _fire():
                    pltpu.async_copy(
                        x_hbm.at[src_all.at[pl.ds(t * CHUNK, CHUNK)]],
                        rows[b], sems[b])

                cc = t - LOOKAHEAD
                jc = (b - LOOKAHEAD) % NSLOT

                @pl.when((cc >= 0) & (cc < NCHUNK))
                def _consume():
                    @pl.when((cc > 0) & (cc % QCH == 0))
                    def _reload_dst():
                        pltpu.sync_copy(
                            dst_hbm.at[wid,
                                       pl.ds(pl.multiple_of(cc, QCH), QCH)],
                            dst_all)
                    cq = cc % QCH
                    pltpu.make_async_copy(dummy, rows[jc], sems[jc]).wait()
                    pltpu.sync_copy(rows[jc], acc.at[dst_all.at[cq]],
                                    add=True)
                    if with_deg:
                        pltpu.sync_copy(ones_v, dacc.at[dst_all.at[cq]],
                                        add=True)
            return carry
        lax.fori_loop(0, NGROUP, pipe_group, 0)
        plsc.subcore_barrier()

        out0 = c * NP + s * ROWS_PER_TILE
        pltpu.sync_copy(acc.at[pl.ds(s * ROWS_PER_TILE, ROWS_PER_TILE)],
                        sum_hbm.at[pl.ds(out0, ROWS_PER_TILE)])
        if with_deg:
            pltpu.sync_copy(dacc.at[pl.ds(s * ROWS_PER_TILE, ROWS_PER_TILE)],
                            deg_hbm.at[pl.ds(out0, ROWS_PER_TILE)])

    return pl.kernel(body, out_type=tuple(out_type), mesh=mesh,
                     scratch_types=tuple(scratch))


BLK = 1024
NBLK = NP // BLK
_DN = (((1,), (1,)), ((), ()))


def _tc1_body(p0_ref, p1_ref, d0_ref, d1_ref, x_ref, wl_ref, bl_ref, wr_ref,
              hr_ref, s1_ref, s2_ref, a1, a2):
    i = pl.program_id(0)
    summed = p0_ref[...] + p1_ref[...]
    deg = jnp.maximum(d0_ref[...] + d1_ref[...], 1.0)
    mean = summed / deg
    out = (lax.dot_general(mean, wl_ref[...], _DN,
                           preferred_element_type=jnp.float32)
           + bl_ref[...]
           + lax.dot_general(x_ref[...], wr_ref[...], _DN,
                             preferred_element_type=jnp.float32))
    nrm = jnp.sqrt(jnp.sum(out * out, axis=1, keepdims=True))
    hr = jnp.maximum(out / jnp.maximum(nrm, 1e-12), 0.0)
    hr_ref[...] = hr
    rowid = i * BLK + lax.broadcasted_iota(jnp.int32, (BLK, 1), 0)
    hrm = jnp.where(rowid < N, hr, 0.0)
    bs1 = jnp.sum(hrm, axis=0, keepdims=True)
    bs2 = jnp.sum(hrm * hrm, axis=0, keepdims=True)

    @pl.when(i == 0)
    def _init():
        a1[...] = bs1
        a2[...] = bs2

    @pl.when(i > 0)
    def _accum():
        a1[...] += bs1
        a2[...] += bs2

    @pl.when(i == NBLK - 1)
    def _emit():
        s1_ref[...] = a1[...]
        s2_ref[...] = a2[...]


def _tc2_body(q0_ref, q1_ref, d0_ref, d1_ref, hr_ref, s1_ref, s2_ref,
              g_ref, be_ref, wl_ref, bl_ref, wr_ref, out_ref):
    mu = s1_ref[...] * (1.0 / N)
    var = s2_ref[...] * (1.0 / N) - mu * mu
    a = g_ref[...] * lax.rsqrt(var + 1e-5)
    bb = be_ref[...] - mu * a
    sumb = q0_ref[...] + q1_ref[...]
    degr = d0_ref[...] + d1_ref[...]
    deg = jnp.maximum(degr, 1.0)
    meanh = jnp.where(degr > 0.5, a * (sumb / deg) + bb, 0.0)
    h = a * hr_ref[...] + bb
    out = (lax.dot_general(meanh, wl_ref[...], _DN,
                           preferred_element_type=jnp.float32)
           + bl_ref[...]
           + lax.dot_general(h, wr_ref[...], _DN,
                             preferred_element_type=jnp.float32))
    nrm = jnp.sqrt(jnp.sum(out * out, axis=1, keepdims=True))
    out_ref[...] = out / jnp.maximum(nrm, 1e-12)


def _row_spec(nrow=BLK):
    return pl.BlockSpec((nrow, D), lambda i: (i, 0))


def _full_spec(shape):
    return pl.BlockSpec(shape, lambda i: tuple(0 for _ in shape))


def _tc1(sum2, d0, d1, x_p, w_l, b_l, w_r):
    return pl.pallas_call(
        _tc1_body,
        grid=(NBLK,),
        in_specs=[
            pl.BlockSpec((BLK, D), lambda i: (i, 0)),
            pl.BlockSpec((BLK, D), lambda i: (i + NBLK, 0)),
            pl.BlockSpec((BLK, 1), lambda i: (i, 0)),
            pl.BlockSpec((BLK, 1), lambda i: (i, 0)),
            pl.BlockSpec((BLK, D), lambda i: (i, 0)),
            _full_spec((D, D)),
            _full_spec((1, D)),
            _full_spec((D, D)),
        ],
        out_specs=[
            pl.BlockSpec((BLK, D), lambda i: (i, 0)),
            _full_spec((1, D)),
            _full_spec((1, D)),
        ],
        out_shape=[
            jax.ShapeDtypeStruct((NP, D), jnp.float32),
            jax.ShapeDtypeStruct((1, D), jnp.float32),
            jax.ShapeDtypeStruct((1, D), jnp.float32),
        ],
        scratch_shapes=[
            pltpu.VMEM((1, D), jnp.float32),
            pltpu.VMEM((1, D), jnp.float32),
        ],
    )(sum2, sum2, d0, d1, x_p, w_l, b_l, w_r)


def _tc2(sum2b, d0, d1, hr, s1, s2, g, be, w_l, b_l, w_r):
    return pl.pallas_call(
        _tc2_body,
        grid=(NBLK,),
        in_specs=[
            pl.BlockSpec((BLK, D), lambda i: (i, 0)),
            pl.BlockSpec((BLK, D), lambda i: (i + NBLK, 0)),
            pl.BlockSpec((BLK, 1), lambda i: (i, 0)),
            pl.BlockSpec((BLK, 1), lambda i: (i, 0)),
            pl.BlockSpec((BLK, D), lambda i: (i, 0)),
            _full_spec((1, D)),
            _full_spec((1, D)),
            _full_spec((1, D)),
            _full_spec((1, D)),
            _full_spec((D, D)),
            _full_spec((1, D)),
            _full_spec((D, D)),
        ],
        out_specs=pl.BlockSpec((BLK, D), lambda i: (i, 0)),
        out_shape=jax.ShapeDtypeStruct((N, D), jnp.float32),
    )(sum2b, sum2b, d0, d1, hr, s1, s2, g, be, w_l, b_l, w_r)


def kernel(x, edge_index, W1_l, b1_l, W1_r, gamma, beta, W2_l, b2_l, W2_r):
    src = jnp.concatenate(
        [edge_index[0], jnp.zeros((EPAD - E,), edge_index.dtype)])
    dst = jnp.concatenate(
        [edge_index[1], jnp.full((EPAD - E,), NP - 1, edge_index.dtype)])
    dst = dst.reshape(NW, NCHUNK, CHUNK)
    sum2, deg2 = _make_sc_kernel(True)(x, src, dst)
    d0 = deg2[0:NP].reshape(NP, 1)
    d1 = deg2[NP:].reshape(NP, 1)
    hr, s1, s2 = _tc1(sum2, d0, d1, x, W1_l, b1_l.reshape(1, D), W1_r)
    sum2b = _make_sc_kernel(False)(hr, src, dst)
    if isinstance(sum2b, (tuple, list)):
        (sum2b,) = sum2b
    out = _tc2(sum2b, d0, d1, hr, s1, s2, gamma.reshape(1, D),
               beta.reshape(1, D), W2_l, b2_l.reshape(1, D), W2_r)
    return out


# speedup vs baseline: 3.3678x; 3.3678x over previous
import functools

import jax
import jax.numpy as jnp
from jax import lax
from jax.experimental import pallas as pl
from jax.experimental.pallas import tpu as pltpu
from jax.experimental.pallas import tpu_sc as plsc

N = 10000
E = 320000
D = 128

NC = 2
NS = 16
NW = NC * NS
EPW = E // NW
CHUNK = 80
NCHUNK = EPW // CHUNK
NP = 10240
ROWS_PER_TILE = NP // NS
ZROWS = 128
NSLOT = 2
LOOKAHEAD = 1


@functools.cache
def _make_sc_kernel(with_deg):
    out_type = [jax.ShapeDtypeStruct((NC * NP, D), jnp.float32)]
    scratch = [
        pltpu.VMEM_SHARED((NP, D), jnp.float32),
        pltpu.VMEM((EPW,), jnp.int32),
        pltpu.VMEM((NCHUNK, CHUNK), jnp.int32),
        [pltpu.VMEM((CHUNK, D), jnp.float32) for _ in range(NSLOT)],
        [pltpu.SemaphoreType.DMA for _ in range(NSLOT)],
    ]
    if with_deg:
        out_type.append(jax.ShapeDtypeStruct((NC * NP,), jnp.float32))
        scratch += [
            pltpu.VMEM_SHARED((NP,), jnp.float32),
            pltpu.VMEM((CHUNK,), jnp.float32),
            pltpu.VMEM((ROWS_PER_TILE,), jnp.float32),
        ]

    mesh = plsc.VectorSubcoreMesh(core_axis_name="c", subcore_axis_name="s",
                                  num_cores=NC, num_subcores=NS)

    def body(*refs):
        if with_deg:
            (x_hbm, src_hbm, dst_hbm, sum_hbm, deg_hbm,
             acc, src_all, dst_all, rows, sems,
             dacc, ones_v, dzbuf) = refs
        else:
            (x_hbm, src_hbm, dst_hbm, sum_hbm,
             acc, src_all, dst_all, rows, sems) = refs
        zbuf = rows[0]
        c = lax.axis_index("c")
        s = lax.axis_index("s")
        wid = c * NS + s

        def zfill(i, carry):
            r = i // (D // 16)
            col = (i % (D // 16)) * 16
            zbuf[r, pl.ds(col, 16)] = jnp.zeros((16,), jnp.float32)
            return carry
        lax.fori_loop(0, CHUNK * (D // 16), zfill, 0)
        if with_deg:
            def ofill(i, carry):
                ones_v[pl.ds(i * 16, 16)] = jnp.ones((16,), jnp.float32)
                return carry
            lax.fori_loop(0, CHUNK // 16, ofill, 0)

            def dzfill(i, carry):
                dzbuf[pl.ds(i * 16, 16)] = jnp.zeros((16,), jnp.float32)
                return carry
            lax.fori_loop(0, ROWS_PER_TILE // 16, dzfill, 0)

        row0 = s * ROWS_PER_TILE

        def zcopy(i, carry):
            pltpu.sync_copy(zbuf, acc.at[pl.ds(row0 + i * CHUNK, CHUNK)])
            return carry
        lax.fori_loop(0, ROWS_PER_TILE // CHUNK, zcopy, 0)
        if with_deg:
            pltpu.sync_copy(dzbuf, dacc.at[pl.ds(row0, ROWS_PER_TILE)])
        plsc.subcore_barrier()

        pltpu.sync_copy(src_hbm.at[pl.ds(wid * EPW, EPW)], src_all)
        pltpu.sync_copy(dst_hbm.at[wid], dst_all)

        dummy = x_hbm.at[pl.ds(0, CHUNK)]
        dummy_deg = x_hbm.at[0, pl.ds(0, CHUNK)]
        NGROUP = (NCHUNK + LOOKAHEAD + NSLOT - 1) // NSLOT

        def pipe_group(g, carry):
            for b in range(NSLOT):
                t = g * NSLOT + b

                @pl.when(t < NCHUNK)
                def _fire():
                    pltpu.async_copy(
                        x_hbm.at[src_all.at[pl.ds(t * CHUNK, CHUNK)]],
                        rows[b], sems[b])

                cc = t - LOOKAHEAD
                jc = (b - LOOKAHEAD) % NSLOT

                @pl.when((cc >= 0) & (cc < NCHUNK))
                def _consume():
                    pltpu.make_async_copy(dummy, rows[jc], sems[jc]).wait()
                    pltpu.sync_copy(rows[jc], acc.at[dst_all.at[cc]],
                                    add=True)
                    if with_deg:
                        pltpu.sync_copy(ones_v, dacc.at[dst_all.at[cc]],
                                        add=True)
            return carry
        lax.fori_loop(0, NGROUP, pipe_group, 0)
        plsc.subcore_barrier()

        out0 = c * NP + s * ROWS_PER_TILE
        pltpu.sync_copy(acc.at[pl.ds(s * ROWS_PER_TILE, ROWS_PER_TILE)],
                        sum_hbm.at[pl.ds(out0, ROWS_PER_TILE)])
        if with_deg:
            pltpu.sync_copy(dacc.at[pl.ds(s * ROWS_PER_TILE, ROWS_PER_TILE)],
                            deg_hbm.at[pl.ds(out0, ROWS_PER_TILE)])

    return pl.kernel(body, out_type=tuple(out_type), mesh=mesh,
                     scratch_types=tuple(scratch))


BLK = 1024
NBLK = NP // BLK
_DN = (((1,), (1,)), ((), ()))


def _tc1_body(p0_ref, p1_ref, d0_ref, d1_ref, x_ref, wl_ref, bl_ref, wr_ref,
              hr_ref, s1_ref, s2_ref, a1, a2):
    i = pl.program_id(0)
    summed = p0_ref[...] + p1_ref[...]
    deg = jnp.maximum(d0_ref[...] + d1_ref[...], 1.0)
    mean = summed / deg
    out = (lax.dot_general(mean, wl_ref[...], _DN,
                           preferred_element_type=jnp.float32)
           + bl_ref[...]
           + lax.dot_general(x_ref[...], wr_ref[...], _DN,
                             preferred_element_type=jnp.float32))
    nrm = jnp.sqrt(jnp.sum(out * out, axis=1, keepdims=True))
    hr = jnp.maximum(out / jnp.maximum(nrm, 1e-12), 0.0)
    hr_ref[...] = hr
    rowid = i * BLK + lax.broadcasted_iota(jnp.int32, (BLK, 1), 0)
    hrm = jnp.where(rowid < N, hr, 0.0)
    bs1 = jnp.sum(hrm, axis=0, keepdims=True)
    bs2 = jnp.sum(hrm * hrm, axis=0, keepdims=True)

    @pl.when(i == 0)
    def _init():
        a1[...] = bs1
        a2[...] = bs2

    @pl.when(i > 0)
    def _accum():
        a1[...] += bs1
        a2[...] += bs2

    @pl.when(i == NBLK - 1)
    def _emit():
        s1_ref[...] = a1[...]
        s2_ref[...] = a2[...]


def _tc2_body(q0_ref, q1_ref, d0_ref, d1_ref, hr_ref, s1_ref, s2_ref,
              g_ref, be_ref, wl_ref, bl_ref, wr_ref, out_ref):
    mu = s1_ref[...] * (1.0 / N)
    var = s2_ref[...] * (1.0 / N) - mu * mu
    a = g_ref[...] * lax.rsqrt(var + 1e-5)
    bb = be_ref[...] - mu * a
    sumb = q0_ref[...] + q1_ref[...]
    degr = d0_ref[...] + d1_ref[...]
    deg = jnp.maximum(degr, 1.0)
    meanh = jnp.where(degr > 0.5, a * (sumb / deg) + bb, 0.0)
    h = a * hr_ref[...] + bb
    out = (lax.dot_general(meanh, wl_ref[...], _DN,
                           preferred_element_type=jnp.float32)
           + bl_ref[...]
           + lax.dot_general(h, wr_ref[...], _DN,
                             preferred_element_type=jnp.float32))
    nrm = jnp.sqrt(jnp.sum(out * out, axis=1, keepdims=True))
    out_ref[...] = out / jnp.maximum(nrm, 1e-12)


def _row_spec(nrow=BLK):
    return pl.BlockSpec((nrow, D), lambda i: (i, 0))


def _full_spec(shape):
    return pl.BlockSpec(shape, lambda i: tuple(0 for _ in shape))


def _tc1(sum2, d0, d1, x_p, w_l, b_l, w_r):
    return pl.pallas_call(
        _tc1_body,
        grid=(NBLK,),
        in_specs=[
            pl.BlockSpec((BLK, D), lambda i: (i, 0)),
            pl.BlockSpec((BLK, D), lambda i: (i + NBLK, 0)),
            pl.BlockSpec((BLK, 1), lambda i: (i, 0)),
            pl.BlockSpec((BLK, 1), lambda i: (i, 0)),
            pl.BlockSpec((BLK, D), lambda i: (i, 0)),
            _full_spec((D, D)),
            _full_spec((1, D)),
            _full_spec((D, D)),
        ],
        out_specs=[
            pl.BlockSpec((BLK, D), lambda i: (i, 0)),
            _full_spec((1, D)),
            _full_spec((1, D)),
        ],
        out_shape=[
            jax.ShapeDtypeStruct((NP, D), jnp.float32),
            jax.ShapeDtypeStruct((1, D), jnp.float32),
            jax.ShapeDtypeStruct((1, D), jnp.float32),
        ],
        scratch_shapes=[
            pltpu.VMEM((1, D), jnp.float32),
            pltpu.VMEM((1, D), jnp.float32),
        ],
    )(sum2, sum2, d0, d1, x_p, w_l, b_l, w_r)


def _tc2(sum2b, d0, d1, hr, s1, s2, g, be, w_l, b_l, w_r):
    return pl.pallas_call(
        _tc2_body,
        grid=(NBLK,),
        in_specs=[
            pl.BlockSpec((BLK, D), lambda i: (i, 0)),
            pl.BlockSpec((BLK, D), lambda i: (i + NBLK, 0)),
            pl.BlockSpec((BLK, 1), lambda i: (i, 0)),
            pl.BlockSpec((BLK, 1), lambda i: (i, 0)),
            pl.BlockSpec((BLK, D), lambda i: (i, 0)),
            _full_spec((1, D)),
            _full_spec((1, D)),
            _full_spec((1, D)),
            _full_spec((1, D)),
            _full_spec((D, D)),
            _full_spec((1, D)),
            _full_spec((D, D)),
        ],
        out_specs=pl.BlockSpec((BLK, D), lambda i: (i, 0)),
        out_shape=jax.ShapeDtypeStruct((N, D), jnp.float32),
    )(sum2b, sum2b, d0, d1, hr, s1, s2, g, be, w_l, b_l, w_r)


def kernel(x, edge_index, W1_l, b1_l, W1_r, gamma, beta, W2_l, b2_l, W2_r):
    src = edge_index[0]
    dst = edge_index[1].reshape(NW, NCHUNK, CHUNK)
    sum2, deg2 = _make_sc_kernel(True)(x, src, dst)
    d0 = deg2[0:NP].reshape(NP, 1)
    d1 = deg2[NP:].reshape(NP, 1)
    hr, s1, s2 = _tc1(sum2, d0, d1, x, W1_l, b1_l.reshape(1, D), W1_r)
    sum2b = _make_sc_kernel(False)(hr, src, dst)
    if isinstance(sum2b, (tuple, list)):
        (sum2b,) = sum2b
    out = _tc2(sum2b, d0, d1, hr, s1, s2, gamma.reshape(1, D),
               beta.reshape(1, D), W2_l, b2_l.reshape(1, D), W2_r)
    return out


# speedup vs baseline: 3.4086x; 1.0121x over previous
import functools

import jax
import jax.numpy as jnp
from jax import lax
from jax.experimental import pallas as pl
from jax.experimental.pallas import tpu as pltpu
from jax.experimental.pallas import tpu_sc as plsc

N = 10000
E = 320000
D = 128

NC = 2
NS = 16
NW = NC * NS
EPW = E // NW
CHUNK = 80
NCHUNK = EPW // CHUNK
NP = 10240
ROWS_PER_TILE = NP // NS
ZROWS = 128


@functools.cache
def _make_sc_kernel(with_deg):
    NSLOT = 2
    LOOKAHEAD = 1
    out_type = [jax.ShapeDtypeStruct((NC * NP, D), jnp.float32)]
    scratch = [
        pltpu.VMEM_SHARED((NP, D), jnp.float32),
        pltpu.VMEM((EPW,), jnp.int32),
        pltpu.VMEM((EPW,), jnp.int32),
        [pltpu.VMEM((CHUNK, D), jnp.float32) for _ in range(NSLOT)],
        [pltpu.SemaphoreType.DMA for _ in range(NSLOT)],
    ]
    if with_deg:
        out_type.append(jax.ShapeDtypeStruct((NC * NP,), jnp.float32))
        scratch += [
            pltpu.VMEM_SHARED((NP,), jnp.float32),
            pltpu.VMEM((CHUNK,), jnp.float32),
            pltpu.VMEM((ROWS_PER_TILE,), jnp.float32),
        ]

    mesh = plsc.VectorSubcoreMesh(core_axis_name="c", subcore_axis_name="s",
                                  num_cores=NC, num_subcores=NS)

    def body(*refs):
        if with_deg:
            (x_hbm, src_hbm, dst_hbm, sum_hbm, deg_hbm,
             acc, src_all, dst_all, rows, sems,
             dacc, ones_v, dzbuf) = refs
        else:
            (x_hbm, src_hbm, dst_hbm, sum_hbm,
             acc, src_all, dst_all, rows, sems) = refs
        zbuf = rows[0]
        c = lax.axis_index("c")
        s = lax.axis_index("s")
        wid = c * NS + s

        def zfill(i, carry):
            r = i // (D // 16)
            col = (i % (D // 16)) * 16
            zbuf[r, pl.ds(col, 16)] = jnp.zeros((16,), jnp.float32)
            return carry
        lax.fori_loop(0, CHUNK * (D // 16), zfill, 0)
        if with_deg:
            def ofill(i, carry):
                ones_v[pl.ds(i * 16, 16)] = jnp.ones((16,), jnp.float32)
                return carry
            lax.fori_loop(0, CHUNK // 16, ofill, 0)

            def dzfill(i, carry):
                dzbuf[pl.ds(i * 16, 16)] = jnp.zeros((16,), jnp.float32)
                return carry
            lax.fori_loop(0, ROWS_PER_TILE // 16, dzfill, 0)

        row0 = s * ROWS_PER_TILE

        def zcopy(i, carry):
            pltpu.sync_copy(zbuf, acc.at[pl.ds(row0 + i * CHUNK, CHUNK)])
            return carry
        lax.fori_loop(0, ROWS_PER_TILE // CHUNK, zcopy, 0)
        if with_deg:
            pltpu.sync_copy(dzbuf, dacc.at[pl.ds(row0, ROWS_PER_TILE)])
        plsc.subcore_barrier()

        pltpu.sync_copy(src_hbm.at[pl.ds(wid * EPW, EPW)], src_all)
        pltpu.sync_copy(dst_hbm.at[pl.ds(wid * EPW, EPW)], dst_all)

        dummy = x_hbm.at[pl.ds(0, CHUNK)]
        dummy_deg = x_hbm.at[0, pl.ds(0, CHUNK)]
        NGROUP = (NCHUNK + LOOKAHEAD + NSLOT - 1) // NSLOT

        def pipe_group(g, carry):
            for b in range(NSLOT):
                t = g * NSLOT + b

                @pl.when(t < NCHUNK)
                def _fire():
                    pltpu.async_copy(
                        x_hbm.at[src_all.at[pl.ds(t * CHUNK, CHUNK)]],
                        rows[b], sems[b])

                cc = t - LOOKAHEAD
                jc = (b - LOOKAHEAD) % NSLOT

                @pl.when((cc >= 0) & (cc < NCHUNK))
                def _consume():
                    didx = dst_all.at[pl.ds(cc * CHUNK, CHUNK)]
                    pltpu.make_async_copy(dummy, rows[jc], sems[jc]).wait()
                    pltpu.sync_copy(rows[jc], acc.at[didx], add=True)
                    if with_deg:
                        pltpu.sync_copy(ones_v, dacc.at[didx], add=True)
            return carry
        lax.fori_loop(0, NGROUP, pipe_group, 0)
        plsc.subcore_barrier()

        out0 = c * NP + s * ROWS_PER_TILE
        pltpu.sync_copy(acc.at[pl.ds(s * ROWS_PER_TILE, ROWS_PER_TILE)],
                        sum_hbm.at[pl.ds(out0, ROWS_PER_TILE)])
        if with_deg:
            pltpu.sync_copy(dacc.at[pl.ds(s * ROWS_PER_TILE, ROWS_PER_TILE)],
                            deg_hbm.at[pl.ds(out0, ROWS_PER_TILE)])

    return pl.kernel(body, out_type=tuple(out_type), mesh=mesh,
                     scratch_types=tuple(scratch))


BLK = 1024
NBLK = NP // BLK
_DN = (((1,), (1,)), ((), ()))


def _tc1_body(p0_ref, p1_ref, d0_ref, d1_ref, x_ref, wl_ref, bl_ref, wr_ref,
              hr_ref, s1_ref, s2_ref, a1, a2):
    i = pl.program_id(0)
    summed = p0_ref[...] + p1_ref[...]
    deg = jnp.maximum(d0_ref[...] + d1_ref[...], 1.0)
    mean = summed / deg
    out = (lax.dot_general(mean, wl_ref[...], _DN,
                           preferred_element_type=jnp.float32)
           + bl_ref[...]
           + lax.dot_general(x_ref[...], wr_ref[...], _DN,
                             preferred_element_type=jnp.float32))
    nrm = jnp.sqrt(jnp.sum(out * out, axis=1, keepdims=True))
    hr = jnp.maximum(out / jnp.maximum(nrm, 1e-12), 0.0)
    hr_ref[...] = hr
    rowid = i * BLK + lax.broadcasted_iota(jnp.int32, (BLK, 1), 0)
    hrm = jnp.where(rowid < N, hr, 0.0)
    bs1 = jnp.sum(hrm, axis=0, keepdims=True)
    bs2 = jnp.sum(hrm * hrm, axis=0, keepdims=True)

    @pl.when(i == 0)
    def _init():
        a1[...] = bs1
        a2[...] = bs2

    @pl.when(i > 0)
    def _accum():
        a1[...] += bs1
        a2[...] += bs2

    @pl.when(i == NBLK - 1)
    def _emit():
        s1_ref[...] = a1[...]
        s2_ref[...] = a2[...]


def _tc2_body(q0_ref, q1_ref, d0_ref, d1_ref, hr_ref, s1_ref, s2_ref,
              g_ref, be_ref, wl_ref, bl_ref, wr_ref, out_ref):
    mu = s1_ref[...] * (1.0 / N)
    var = s2_ref[...] * (1.0 / N) - mu * mu
    a = g_ref[...] * lax.rsqrt(var + 1e-5)
    bb = be_ref[...] - mu * a
    sumb = q0_ref[...] + q1_ref[...]
    degr = d0_ref[...] + d1_ref[...]
    deg = jnp.maximum(degr, 1.0)
    meanh = jnp.where(degr > 0.5, a * (sumb / deg) + bb, 0.0)
    h = a * hr_ref[...] + bb
    out = (lax.dot_general(meanh, wl_ref[...], _DN,
                           preferred_element_type=jnp.float32)
           + bl_ref[...]
           + lax.dot_general(h, wr_ref[...], _DN,
                             preferred_element_type=jnp.float32))
    nrm = jnp.sqrt(jnp.sum(out * out, axis=1, keepdims=True))
    out_ref[...] = out / jnp.maximum(nrm, 1e-12)


def _row_spec(nrow=BLK):
    return pl.BlockSpec((nrow, D), lambda i: (i, 0))


def _full_spec(shape):
    return pl.BlockSpec(shape, lambda i: tuple(0 for _ in shape))


def _tc1(sum2, d0, d1, x_p, w_l, b_l, w_r):
    return pl.pallas_call(
        _tc1_body,
        grid=(NBLK,),
        in_specs=[
            pl.BlockSpec((BLK, D), lambda i: (i, 0)),
            pl.BlockSpec((BLK, D), lambda i: (i + NBLK, 0)),
            pl.BlockSpec((BLK, 1), lambda i: (i, 0)),
            pl.BlockSpec((BLK, 1), lambda i: (i, 0)),
            pl.BlockSpec((BLK, D), lambda i: (i, 0)),
            _full_spec((D, D)),
            _full_spec((1, D)),
            _full_spec((D, D)),
        ],
        out_specs=[
            pl.BlockSpec((BLK, D), lambda i: (i, 0)),
            _full_spec((1, D)),
            _full_spec((1, D)),
        ],
        out_shape=[
            jax.ShapeDtypeStruct((NP, D), jnp.float32),
            jax.ShapeDtypeStruct((1, D), jnp.float32),
            jax.ShapeDtypeStruct((1, D), jnp.float32),
        ],
        scratch_shapes=[
            pltpu.VMEM((1, D), jnp.float32),
            pltpu.VMEM((1, D), jnp.float32),
        ],
    )(sum2, sum2, d0, d1, x_p, w_l, b_l, w_r)


def _tc2(sum2b, d0, d1, hr, s1, s2, g, be, w_l, b_l, w_r):
    return pl.pallas_call(
        _tc2_body,
        grid=(NBLK,),
        in_specs=[
            pl.BlockSpec((BLK, D), lambda i: (i, 0)),
            pl.BlockSpec((BLK, D), lambda i: (i + NBLK, 0)),
            pl.BlockSpec((BLK, 1), lambda i: (i, 0)),
            pl.BlockSpec((BLK, 1), lambda i: (i, 0)),
            pl.BlockSpec((BLK, D), lambda i: (i, 0)),
            _full_spec((1, D)),
            _full_spec((1, D)),
            _full_spec((1, D)),
            _full_spec((1, D)),
            _full_spec((D, D)),
            _full_spec((1, D)),
            _full_spec((D, D)),
        ],
        out_specs=pl.BlockSpec((BLK, D), lambda i: (i, 0)),
        out_shape=jax.ShapeDtypeStruct((N, D), jnp.float32),
    )(sum2b, sum2b, d0, d1, hr, s1, s2, g, be, w_l, b_l, w_r)


def kernel(x, edge_index, W1_l, b1_l, W1_r, gamma, beta, W2_l, b2_l, W2_r):
    src = edge_index[0]
    dst = edge_index[1]
    sum2, deg2 = _make_sc_kernel(True)(x, src, dst)
    d0 = deg2[0:NP].reshape(NP, 1)
    d1 = deg2[NP:].reshape(NP, 1)
    hr, s1, s2 = _tc1(sum2, d0, d1, x, W1_l, b1_l.reshape(1, D), W1_r)
    sum2b = _make_sc_kernel(False)(hr, src, dst)
    if isinstance(sum2b, (tuple, list)):
        (sum2b,) = sum2b
    out = _tc2(sum2b, d0, d1, hr, s1, s2, gamma.reshape(1, D),
               beta.reshape(1, D), W2_l, b2_l.reshape(1, D), W2_r)
    return out


# speedup vs baseline: 3.4393x; 1.0090x over previous
import functools

import jax
import jax.numpy as jnp
from jax import lax
from jax.experimental import pallas as pl
from jax.experimental.pallas import tpu as pltpu
from jax.experimental.pallas import tpu_sc as plsc

N = 10000
E = 320000
D = 128

NC = 2
NS = 16
NW = NC * NS
EPW = E // NW
CHUNK = 80
NCHUNK = EPW // CHUNK
NP = 10240
ROWS_PER_TILE = NP // NS
ZROWS = 128


@functools.cache
def _make_sc_kernel(with_deg):
    NSLOT = 2
    LOOKAHEAD = 1
    out_type = [jax.ShapeDtypeStruct((NC * NP, D), jnp.float32)]
    scratch = [
        pltpu.VMEM_SHARED((NP, D), jnp.float32),
        pltpu.VMEM((EPW,), jnp.int32),
        pltpu.VMEM((EPW,), jnp.int32),
        [pltpu.VMEM((CHUNK, D), jnp.float32) for _ in range(NSLOT)],
        [pltpu.SemaphoreType.DMA for _ in range(NSLOT)],
    ]
    if with_deg:
        out_type.append(jax.ShapeDtypeStruct((NC * NP,), jnp.float32))
        scratch += [
            pltpu.VMEM_SHARED((NP,), jnp.float32),
            pltpu.VMEM((CHUNK,), jnp.float32),
            pltpu.VMEM((ROWS_PER_TILE,), jnp.float32),
        ]

    mesh = plsc.VectorSubcoreMesh(core_axis_name="c", subcore_axis_name="s",
                                  num_cores=NC, num_subcores=NS)

    def body(*refs):
        if with_deg:
            (x_hbm, src_hbm, dst_hbm, sum_hbm, deg_hbm,
             acc, src_all, dst_all, rows, sems,
             dacc, ones_v, dzbuf) = refs
        else:
            (x_hbm, src_hbm, dst_hbm, sum_hbm,
             acc, src_all, dst_all, rows, sems) = refs
        zbuf = rows[0]
        c = lax.axis_index("c")
        s = lax.axis_index("s")
        wid = c * NS + s

        def zfill(i, carry):
            r = i // (D // 16)
            col = (i % (D // 16)) * 16
            zbuf[r, pl.ds(col, 16)] = jnp.zeros((16,), jnp.float32)
            return carry
        lax.fori_loop(0, CHUNK * (D // 16), zfill, 0)
        if with_deg:
            def ofill(i, carry):
                ones_v[pl.ds(i * 16, 16)] = jnp.ones((16,), jnp.float32)
                return carry
            lax.fori_loop(0, CHUNK // 16, ofill, 0)

            def dzfill(i, carry):
                dzbuf[pl.ds(i * 16, 16)] = jnp.zeros((16,), jnp.float32)
                return carry
            lax.fori_loop(0, ROWS_PER_TILE // 16, dzfill, 0)

        row0 = s * ROWS_PER_TILE

        def zcopy(i, carry):
            pltpu.sync_copy(zbuf, acc.at[pl.ds(row0 + i * CHUNK, CHUNK)])
            return carry
        lax.fori_loop(0, ROWS_PER_TILE // CHUNK, zcopy, 0)
        if with_deg:
            pltpu.sync_copy(dzbuf, dacc.at[pl.ds(row0, ROWS_PER_TILE)])
        plsc.subcore_barrier()

        pltpu.sync_copy(src_hbm.at[pl.ds(wid * EPW, EPW)], src_all)
        pltpu.sync_copy(dst_hbm.at[pl.ds(wid * EPW, EPW)], dst_all)

        dummy = x_hbm.at[pl.ds(0, CHUNK)]
        dummy_deg = x_hbm.at[0, pl.ds(0, CHUNK)]
        NGROUP = (NCHUNK + LOOKAHEAD + NSLOT - 1) // NSLOT

        def pipe_group(g, carry):
            for b in range(NSLOT):
                t = g * NSLOT + b

                @pl.when(t < NCHUNK)
                def _fire():
                    pltpu.async_copy(
                        x_hbm.at[src_all.at[pl.ds(t * CHUNK, CHUNK)]],
                        rows[b], sems[b])

                cc = t - LOOKAHEAD
                jc = (b - LOOKAHEAD) % NSLOT

                @pl.when((cc >= 0) & (cc < NCHUNK))
                def _consume():
                    didx = dst_all.at[pl.ds(cc * CHUNK, CHUNK)]
                    pltpu.make_async_copy(dummy, rows[jc], sems[jc]).wait()
                    pltpu.sync_copy(rows[jc], acc.at[didx], add=True)
                    if with_deg:
                        pltpu.sync_copy(ones_v, dacc.at[didx], add=True)
            return carry
        lax.fori_loop(0, NGROUP, pipe_group, 0)
        plsc.subcore_barrier()

        out0 = c * NP + s * ROWS_PER_TILE
        pltpu.sync_copy(acc.at[pl.ds(s * ROWS_PER_TILE, ROWS_PER_TILE)],
                        sum_hbm.at[pl.ds(out0, ROWS_PER_TILE)])
        if with_deg:
            pltpu.sync_copy(dacc.at[pl.ds(s * ROWS_PER_TILE, ROWS_PER_TILE)],
                            deg_hbm.at[pl.ds(out0, ROWS_PER_TILE)])

    return pl.kernel(body, out_type=tuple(out_type), mesh=mesh,
                     scratch_types=tuple(scratch))


BLK = 2048
NBLK = NP // BLK
_DN = (((1,), (1,)), ((), ()))


def _tc1_body(p0_ref, p1_ref, d0_ref, d1_ref, x_ref, wl_ref, bl_ref, wr_ref,
              hr_ref, s1_ref, s2_ref, a1, a2):
    i = pl.program_id(0)
    summed = p0_ref[...] + p1_ref[...]
    deg = jnp.maximum(d0_ref[...] + d1_ref[...], 1.0)
    mean = summed / deg
    out = (lax.dot_general(mean, wl_ref[...], _DN,
                           preferred_element_type=jnp.float32)
           + bl_ref[...]
           + lax.dot_general(x_ref[...], wr_ref[...], _DN,
                             preferred_element_type=jnp.float32))
    nrm = jnp.sqrt(jnp.sum(out * out, axis=1, keepdims=True))
    hr = jnp.maximum(out / jnp.maximum(nrm, 1e-12), 0.0)
    hr_ref[...] = hr
    rowid = i * BLK + lax.broadcasted_iota(jnp.int32, (BLK, 1), 0)
    hrm = jnp.where(rowid < N, hr, 0.0)
    bs1 = jnp.sum(hrm, axis=0, keepdims=True)
    bs2 = jnp.sum(hrm * hrm, axis=0, keepdims=True)

    @pl.when(i == 0)
    def _init():
        a1[...] = bs1
        a2[...] = bs2

    @pl.when(i > 0)
    def _accum():
        a1[...] += bs1
        a2[...] += bs2

    @pl.when(i == NBLK - 1)
    def _emit():
        s1_ref[...] = a1[...]
        s2_ref[...] = a2[...]


def _tc2_body(q0_ref, q1_ref, d0_ref, d1_ref, hr_ref, s1_ref, s2_ref,
              g_ref, be_ref, wl_ref, bl_ref, wr_ref, out_ref):
    mu = s1_ref[...] * (1.0 / N)
    var = s2_ref[...] * (1.0 / N) - mu * mu
    a = g_ref[...] * lax.rsqrt(var + 1e-5)
    bb = be_ref[...] - mu * a
    sumb = q0_ref[...] + q1_ref[...]
    degr = d0_ref[...] + d1_ref[...]
    deg = jnp.maximum(degr, 1.0)
    meanh = jnp.where(degr > 0.5, a * (sumb / deg) + bb, 0.0)
    h = a * hr_ref[...] + bb
    out = (lax.dot_general(meanh, wl_ref[...], _DN,
                           preferred_element_type=jnp.float32)
           + bl_ref[...]
           + lax.dot_general(h, wr_ref[...], _DN,
                             preferred_element_type=jnp.float32))
    nrm = jnp.sqrt(jnp.sum(out * out, axis=1, keepdims=True))
    out_ref[...] = out / jnp.maximum(nrm, 1e-12)


def _row_spec(nrow=BLK):
    return pl.BlockSpec((nrow, D), lambda i: (i, 0))


def _full_spec(shape):
    return pl.BlockSpec(shape, lambda i: tuple(0 for _ in shape))


def _tc1(sum2, d0, d1, x_p, w_l, b_l, w_r):
    return pl.pallas_call(
        _tc1_body,
        grid=(NBLK,),
        in_specs=[
            pl.BlockSpec((BLK, D), lambda i: (i, 0)),
            pl.BlockSpec((BLK, D), lambda i: (i + NBLK, 0)),
            pl.BlockSpec((BLK, 1), lambda i: (i, 0)),
            pl.BlockSpec((BLK, 1), lambda i: (i, 0)),
            pl.BlockSpec((BLK, D), lambda i: (i, 0)),
            _full_spec((D, D)),
            _full_spec((1, D)),
            _full_spec((D, D)),
        ],
        out_specs=[
            pl.BlockSpec((BLK, D), lambda i: (i, 0)),
            _full_spec((1, D)),
            _full_spec((1, D)),
        ],
        out_shape=[
            jax.ShapeDtypeStruct((NP, D), jnp.float32),
            jax.ShapeDtypeStruct((1, D), jnp.float32),
            jax.ShapeDtypeStruct((1, D), jnp.float32),
        ],
        scratch_shapes=[
            pltpu.VMEM((1, D), jnp.float32),
            pltpu.VMEM((1, D), jnp.float32),
        ],
    )(sum2, sum2, d0, d1, x_p, w_l, b_l, w_r)


def _tc2(sum2b, d0, d1, hr, s1, s2, g, be, w_l, b_l, w_r):
    return pl.pallas_call(
        _tc2_body,
        grid=(NBLK,),
        in_specs=[
            pl.BlockSpec((BLK, D), lambda i: (i, 0)),
            pl.BlockSpec((BLK, D), lambda i: (i + NBLK, 0)),
            pl.BlockSpec((BLK, 1), lambda i: (i, 0)),
            pl.BlockSpec((BLK, 1), lambda i: (i, 0)),
            pl.BlockSpec((BLK, D), lambda i: (i, 0)),
            _full_spec((1, D)),
            _full_spec((1, D)),
            _full_spec((1, D)),
            _full_spec((1, D)),
            _full_spec((D, D)),
            _full_spec((1, D)),
            _full_spec((D, D)),
        ],
        out_specs=pl.BlockSpec((BLK, D), lambda i: (i, 0)),
        out_shape=jax.ShapeDtypeStruct((N, D), jnp.float32),
    )(sum2b, sum2b, d0, d1, hr, s1, s2, g, be, w_l, b_l, w_r)


def kernel(x, edge_index, W1_l, b1_l, W1_r, gamma, beta, W2_l, b2_l, W2_r):
    src = edge_index[0]
    dst = edge_index[1]
    sum2, deg2 = _make_sc_kernel(True)(x, src, dst)
    d0 = deg2[0:NP].reshape(NP, 1)
    d1 = deg2[NP:].reshape(NP, 1)
    hr, s1, s2 = _tc1(sum2, d0, d1, x, W1_l, b1_l.reshape(1, D), W1_r)
    sum2b = _make_sc_kernel(False)(hr, src, dst)
    if isinstance(sum2b, (tuple, list)):
        (sum2b,) = sum2b
    out = _tc2(sum2b, d0, d1, hr, s1, s2, gamma.reshape(1, D),
               beta.reshape(1, D), W2_l, b2_l.reshape(1, D), W2_r)
    return out


# speedup vs baseline: 3.4424x; 1.0009x over previous
import functools

import jax
import jax.numpy as jnp
from jax import lax
from jax.experimental import pallas as pl
from jax.experimental.pallas import tpu as pltpu
from jax.experimental.pallas import tpu_sc as plsc

N = 10000
E = 320000
D = 128

NC = 2
NS = 16
NW = NC * NS
EPW = E // NW
CHUNK = 80
NCHUNK = EPW // CHUNK
NP = 10240
ROWS_PER_TILE = NP // NS
ZROWS = 128


@functools.cache
def _make_sc_kernel(with_deg):
    NSLOT = 2
    LOOKAHEAD = 1
    out_type = [jax.ShapeDtypeStruct((NC * NP, D), jnp.float32)]
    scratch = [
        pltpu.VMEM_SHARED((NP, D), jnp.float32),
        pltpu.VMEM((EPW,), jnp.int32),
        pltpu.VMEM((EPW,), jnp.int32),
        [pltpu.VMEM((CHUNK, D), jnp.float32) for _ in range(NSLOT)],
        [pltpu.SemaphoreType.DMA for _ in range(NSLOT)],
    ]
    if with_deg:
        out_type.append(jax.ShapeDtypeStruct((NC * NP,), jnp.float32))
        scratch += [
            pltpu.VMEM_SHARED((NP,), jnp.float32),
            pltpu.VMEM((EPW // 2 + 16,), jnp.float32),
            pltpu.VMEM((ROWS_PER_TILE,), jnp.float32),
        ]

    mesh = plsc.VectorSubcoreMesh(core_axis_name="c", subcore_axis_name="s",
                                  num_cores=NC, num_subcores=NS)

    def body(*refs):
        if with_deg:
            (x_hbm, src_hbm, dst_hbm, sum_hbm, deg_hbm,
             acc, src_all, dst_all, rows, sems,
             dacc, ones_v, dzbuf) = refs
        else:
            (x_hbm, src_hbm, dst_hbm, sum_hbm,
             acc, src_all, dst_all, rows, sems) = refs
        zbuf = rows[0]
        c = lax.axis_index("c")
        s = lax.axis_index("s")
        wid = c * NS + s

        pltpu.sync_copy(src_hbm.at[pl.ds(wid * EPW, EPW)], src_all)
        pltpu.sync_copy(dst_hbm.at[pl.ds(wid * EPW, EPW)], dst_all)

        def zfill(i, carry):
            r = i // (D // 16)
            col = (i % (D // 16)) * 16
            zbuf[r, pl.ds(col, 16)] = jnp.zeros((16,), jnp.float32)
            return carry
        lax.fori_loop(0, CHUNK * (D // 16), zfill, 0)
        if with_deg:
            def ofill(i, carry):
                ones_v[pl.ds(i * 16, 16)] = jnp.ones((16,), jnp.float32)
                return carry
            lax.fori_loop(0, (EPW // 2 + 16) // 16, ofill, 0)

            def dzfill(i, carry):
                dzbuf[pl.ds(i * 16, 16)] = jnp.zeros((16,), jnp.float32)
                return carry
            lax.fori_loop(0, ROWS_PER_TILE // 16, dzfill, 0)

        row0 = s * ROWS_PER_TILE

        def zcopy(i, carry):
            pltpu.sync_copy(zbuf, acc.at[pl.ds(row0 + i * CHUNK, CHUNK)])
            return carry
        lax.fori_loop(0, ROWS_PER_TILE // CHUNK, zcopy, 0)
        if with_deg:
            pltpu.sync_copy(dzbuf, dacc.at[pl.ds(row0, ROWS_PER_TILE)])
        plsc.subcore_barrier()

        if with_deg:
            half = EPW // 2
            for hh in range(2):
                pltpu.sync_copy(
                    ones_v.at[pl.ds(0, half)],
                    dacc.at[dst_all.at[pl.ds(hh * half, half)]],
                    add=True)

        dummy = x_hbm.at[pl.ds(0, CHUNK)]
        NGROUP = (NCHUNK + LOOKAHEAD + NSLOT - 1) // NSLOT

        def pipe_group(g, carry):
            for b in range(NSLOT):
                t = g * NSLOT + b

                @pl.when(t < NCHUNK)
                def _fire():
                    pltpu.async_copy(
                        x_hbm.at[src_all.at[pl.ds(t * CHUNK, CHUNK)]],
                        rows[b], sems[b])

                cc = t - LOOKAHEAD
                jc = (b - LOOKAHEAD) % NSLOT

                @pl.when((cc >= 0) & (cc < NCHUNK))
                def _consume():
                    didx = dst_all.at[pl.ds(cc * CHUNK, CHUNK)]
                    pltpu.make_async_copy(dummy, rows[jc], sems[jc]).wait()
                    pltpu.sync_copy(rows[jc], acc.at[didx], add=True)
            return carry
        lax.fori_loop(0, NGROUP, pipe_group, 0)
        plsc.subcore_barrier()

        out0 = c * NP + s * ROWS_PER_TILE
        pltpu.sync_copy(acc.at[pl.ds(s * ROWS_PER_TILE, ROWS_PER_TILE)],
                        sum_hbm.at[pl.ds(out0, ROWS_PER_TILE)])
        if with_deg:
            pltpu.sync_copy(dacc.at[pl.ds(s * ROWS_PER_TILE, ROWS_PER_TILE)],
                            deg_hbm.at[pl.ds(out0, ROWS_PER_TILE)])

    return pl.kernel(body, out_type=tuple(out_type), mesh=mesh,
                     scratch_types=tuple(scratch))


BLK = 2048
NBLK = NP // BLK
_DN = (((1,), (1,)), ((), ()))


def _tc1_body(p0_ref, p1_ref, d0_ref, d1_ref, x_ref, wl_ref, bl_ref, wr_ref,
              hr_ref, s1_ref, s2_ref, a1, a2):
    i = pl.program_id(0)
    summed = p0_ref[...] + p1_ref[...]
    deg = jnp.maximum(d0_ref[...] + d1_ref[...], 1.0)
    mean = summed / deg
    out = (lax.dot_general(mean, wl_ref[...], _DN,
                           preferred_element_type=jnp.float32)
           + bl_ref[...]
           + lax.dot_general(x_ref[...], wr_ref[...], _DN,
                             preferred_element_type=jnp.float32))
    nrm = jnp.sqrt(jnp.sum(out * out, axis=1, keepdims=True))
    hr = jnp.maximum(out / jnp.maximum(nrm, 1e-12), 0.0)
    hr_ref[...] = hr
    rowid = i * BLK + lax.broadcasted_iota(jnp.int32, (BLK, 1), 0)
    hrm = jnp.where(rowid < N, hr, 0.0)
    bs1 = jnp.sum(hrm, axis=0, keepdims=True)
    bs2 = jnp.sum(hrm * hrm, axis=0, keepdims=True)

    @pl.when(i == 0)
    def _init():
        a1[...] = bs1
        a2[...] = bs2

    @pl.when(i > 0)
    def _accum():
        a1[...] += bs1
        a2[...] += bs2

    @pl.when(i == NBLK - 1)
    def _emit():
        s1_ref[...] = a1[...]
        s2_ref[...] = a2[...]


def _tc2_body(q0_ref, q1_ref, d0_ref, d1_ref, hr_ref, s1_ref, s2_ref,
              g_ref, be_ref, wl_ref, bl_ref, wr_ref, out_ref):
    mu = s1_ref[...] * (1.0 / N)
    var = s2_ref[...] * (1.0 / N) - mu * mu
    a = g_ref[...] * lax.rsqrt(var + 1e-5)
    bb = be_ref[...] - mu * a
    sumb = q0_ref[...] + q1_ref[...]
    degr = d0_ref[...] + d1_ref[...]
    deg = jnp.maximum(degr, 1.0)
    meanh = jnp.where(degr > 0.5, a * (sumb / deg) + bb, 0.0)
    h = a * hr_ref[...] + bb
    out = (lax.dot_general(meanh, wl_ref[...], _DN,
                           preferred_element_type=jnp.float32)
           + bl_ref[...]
           + lax.dot_general(h, wr_ref[...], _DN,
                             preferred_element_type=jnp.float32))
    nrm = jnp.sqrt(jnp.sum(out * out, axis=1, keepdims=True))
    out_ref[...] = out / jnp.maximum(nrm, 1e-12)


def _row_spec(nrow=BLK):
    return pl.BlockSpec((nrow, D), lambda i: (i, 0))


def _full_spec(shape):
    return pl.BlockSpec(shape, lambda i: tuple(0 for _ in shape))


def _tc1(sum2, d0, d1, x_p, w_l, b_l, w_r):
    return pl.pallas_call(
        _tc1_body,
        grid=(NBLK,),
        in_specs=[
            pl.BlockSpec((BLK, D), lambda i: (i, 0)),
            pl.BlockSpec((BLK, D), lambda i: (i + NBLK, 0)),
            pl.BlockSpec((BLK, 1), lambda i: (i, 0)),
            pl.BlockSpec((BLK, 1), lambda i: (i, 0)),
            pl.BlockSpec((BLK, D), lambda i: (i, 0)),
            _full_spec((D, D)),
            _full_spec((1, D)),
            _full_spec((D, D)),
        ],
        out_specs=[
            pl.BlockSpec((BLK, D), lambda i: (i, 0)),
            _full_spec((1, D)),
            _full_spec((1, D)),
        ],
        out_shape=[
            jax.ShapeDtypeStruct((NP, D), jnp.float32),
            jax.ShapeDtypeStruct((1, D), jnp.float32),
            jax.ShapeDtypeStruct((1, D), jnp.float32),
        ],
        scratch_shapes=[
            pltpu.VMEM((1, D), jnp.float32),
            pltpu.VMEM((1, D), jnp.float32),
        ],
    )(sum2, sum2, d0, d1, x_p, w_l, b_l, w_r)


def _tc2(sum2b, d0, d1, hr, s1, s2, g, be, w_l, b_l, w_r):
    return pl.pallas_call(
        _tc2_body,
        grid=(NBLK,),
        in_specs=[
            pl.BlockSpec((BLK, D), lambda i: (i, 0)),
            pl.BlockSpec((BLK, D), lambda i: (i + NBLK, 0)),
            pl.BlockSpec((BLK, 1), lambda i: (i, 0)),
            pl.BlockSpec((BLK, 1), lambda i: (i, 0)),
            pl.BlockSpec((BLK, D), lambda i: (i, 0)),
            _full_spec((1, D)),
            _full_spec((1, D)),
            _full_spec((1, D)),
            _full_spec((1, D)),
            _full_spec((D, D)),
            _full_spec((1, D)),
            _full_spec((D, D)),
        ],
        out_specs=pl.BlockSpec((BLK, D), lambda i: (i, 0)),
        out_shape=jax.ShapeDtypeStruct((N, D), jnp.float32),
    )(sum2b, sum2b, d0, d1, hr, s1, s2, g, be, w_l, b_l, w_r)


def kernel(x, edge_index, W1_l, b1_l, W1_r, gamma, beta, W2_l, b2_l, W2_r):
    src = edge_index[0]
    dst = edge_index[1]
    sum2, deg2 = _make_sc_kernel(True)(x, src, dst)
    d0 = deg2[0:NP].reshape(NP, 1)
    d1 = deg2[NP:].reshape(NP, 1)
    hr, s1, s2 = _tc1(sum2, d0, d1, x, W1_l, b1_l.reshape(1, D), W1_r)
    sum2b = _make_sc_kernel(False)(hr, src, dst)
    if isinstance(sum2b, (tuple, list)):
        (sum2b,) = sum2b
    out = _tc2(sum2b, d0, d1, hr, s1, s2, gamma.reshape(1, D),
               beta.reshape(1, D), W2_l, b2_l.reshape(1, D), W2_r)
    return out


# speedup vs baseline: 3.4827x; 1.0117x over previous
import functools

import jax
import jax.numpy as jnp
from jax import lax
from jax.experimental import pallas as pl
from jax.experimental.pallas import tpu as pltpu
from jax.experimental.pallas import tpu_sc as plsc

N = 10000
E = 320000
D = 128

NC = 2
NS = 16
NW = NC * NS
EPW = E // NW
CHUNK = 80
NCHUNK = EPW // CHUNK
NP = 10240
ROWS_PER_TILE = NP // NS
ZROWS = 128


@functools.cache
def _make_sc_kernel(with_deg):
    NSLOT = 2
    LOOKAHEAD = 1
    out_type = [jax.ShapeDtypeStruct((NC * NP, D), jnp.float32)]
    scratch = [
        pltpu.VMEM_SHARED((NP, D), jnp.float32),
        pltpu.VMEM((EPW,), jnp.int32),
        pltpu.VMEM((EPW,), jnp.int32),
        [pltpu.VMEM((CHUNK, D), jnp.float32) for _ in range(NSLOT)],
        [pltpu.SemaphoreType.DMA for _ in range(NSLOT)],
        pltpu.VMEM((16, D), jnp.float32),
        pltpu.SemaphoreType.DMA,
    ]
    if with_deg:
        out_type.append(jax.ShapeDtypeStruct((NC * NP,), jnp.float32))
        scratch += [
            pltpu.VMEM_SHARED((NP,), jnp.float32),
            pltpu.VMEM((EPW // 2 + 16,), jnp.float32),
            pltpu.VMEM((ROWS_PER_TILE,), jnp.float32),
        ]

    mesh = plsc.VectorSubcoreMesh(core_axis_name="c", subcore_axis_name="s",
                                  num_cores=NC, num_subcores=NS)

    def body(*refs):
        if with_deg:
            (x_hbm, src_hbm, dst_hbm, sum_hbm, deg_hbm,
             acc, src_all, dst_all, rows, sems, zbuf, zsem,
             dacc, ones_v, dzbuf) = refs
        else:
            (x_hbm, src_hbm, dst_hbm, sum_hbm,
             acc, src_all, dst_all, rows, sems, zbuf, zsem) = refs
        c = lax.axis_index("c")
        s = lax.axis_index("s")
        wid = c * NS + s

        pltpu.sync_copy(src_hbm.at[pl.ds(wid * EPW, EPW)], src_all)
        pltpu.sync_copy(dst_hbm.at[pl.ds(wid * EPW, EPW)], dst_all)

        for b in range(NSLOT):
            pltpu.async_copy(
                x_hbm.at[src_all.at[pl.ds(b * CHUNK, CHUNK)]],
                rows[b], sems[b])

        def zfill(i, carry):
            r = i // (D // 16)
            col = (i % (D // 16)) * 16
            zbuf[r, pl.ds(col, 16)] = jnp.zeros((16,), jnp.float32)
            return carry
        lax.fori_loop(0, 16 * (D // 16), zfill, 0)
        if with_deg:
            def ofill(i, carry):
                ones_v[pl.ds(i * 16, 16)] = jnp.ones((16,), jnp.float32)
                return carry
            lax.fori_loop(0, (EPW // 2 + 16) // 16, ofill, 0)

            def dzfill(i, carry):
                dzbuf[pl.ds(i * 16, 16)] = jnp.zeros((16,), jnp.float32)
                return carry
            lax.fori_loop(0, ROWS_PER_TILE // 16, dzfill, 0)

        row0 = s * ROWS_PER_TILE

        def zcopy(i, carry):
            pltpu.sync_copy(zbuf, acc.at[pl.ds(row0 + i * 16, 16)])
            return carry
        lax.fori_loop(0, ROWS_PER_TILE // 16, zcopy, 0)
        if with_deg:
            pltpu.sync_copy(dzbuf, dacc.at[pl.ds(row0, ROWS_PER_TILE)])
        plsc.subcore_barrier()

        if with_deg:
            half = EPW // 2
            for hh in range(2):
                pltpu.sync_copy(
                    ones_v.at[pl.ds(0, half)],
                    dacc.at[dst_all.at[pl.ds(hh * half, half)]],
                    add=True)

        dummy = x_hbm.at[pl.ds(0, CHUNK)]
        NGROUP = (NCHUNK + LOOKAHEAD + NSLOT - 1) // NSLOT

        def pipe_group(g, carry):
            for b in range(NSLOT):
                t = g * NSLOT + b

                @pl.when((t >= NSLOT) & (t < NCHUNK))
                def _fire():
                    pltpu.async_copy(
                        x_hbm.at[src_all.at[pl.ds(t * CHUNK, CHUNK)]],
                        rows[b], sems[b])

                cc = t - LOOKAHEAD
                jc = (b - LOOKAHEAD) % NSLOT

                @pl.when((cc >= 0) & (cc < NCHUNK))
                def _consume():
                    didx = dst_all.at[pl.ds(cc * CHUNK, CHUNK)]
                    pltpu.make_async_copy(dummy, rows[jc], sems[jc]).wait()
                    pltpu.sync_copy(rows[jc], acc.at[didx], add=True)
            return carry
        lax.fori_loop(0, NGROUP, pipe_group, 0)
        plsc.subcore_barrier()

        out0 = c * NP + s * ROWS_PER_TILE
        pltpu.sync_copy(acc.at[pl.ds(s * ROWS_PER_TILE, ROWS_PER_TILE)],
                        sum_hbm.at[pl.ds(out0, ROWS_PER_TILE)])
        if with_deg:
            pltpu.sync_copy(dacc.at[pl.ds(s * ROWS_PER_TILE, ROWS_PER_TILE)],
                            deg_hbm.at[pl.ds(out0, ROWS_PER_TILE)])

    return pl.kernel(body, out_type=tuple(out_type), mesh=mesh,
                     scratch_types=tuple(scratch))


BLK = 2048
NBLK = NP // BLK
_DN = (((1,), (1,)), ((), ()))


def _tc1_body(p0_ref, p1_ref, d0_ref, d1_ref, x_ref, wl_ref, bl_ref, wr_ref,
              hr_ref, s1_ref, s2_ref, a1, a2):
    i = pl.program_id(0)
    summed = p0_ref[...] + p1_ref[...]
    deg = jnp.maximum(d0_ref[...] + d1_ref[...], 1.0)
    mean = summed / deg
    out = (lax.dot_general(mean, wl_ref[...], _DN,
                           preferred_element_type=jnp.float32)
           + bl_ref[...]
           + lax.dot_general(x_ref[...], wr_ref[...], _DN,
                             preferred_element_type=jnp.float32))
    nrm = jnp.sqrt(jnp.sum(out * out, axis=1, keepdims=True))
    hr = jnp.maximum(out / jnp.maximum(nrm, 1e-12), 0.0)
    hr_ref[...] = hr
    rowid = i * BLK + lax.broadcasted_iota(jnp.int32, (BLK, 1), 0)
    hrm = jnp.where(rowid < N, hr, 0.0)
    bs1 = jnp.sum(hrm, axis=0, keepdims=True)
    bs2 = jnp.sum(hrm * hrm, axis=0, keepdims=True)

    @pl.when(i == 0)
    def _init():
        a1[...] = bs1
        a2[...] = bs2

    @pl.when(i > 0)
    def _accum():
        a1[...] += bs1
        a2[...] += bs2

    @pl.when(i == NBLK - 1)
    def _emit():
        s1_ref[...] = a1[...]
        s2_ref[...] = a2[...]


def _tc2_body(q0_ref, q1_ref, d0_ref, d1_ref, hr_ref, s1_ref, s2_ref,
              g_ref, be_ref, wl_ref, bl_ref, wr_ref, out_ref):
    mu = s1_ref[...] * (1.0 / N)
    var = s2_ref[...] * (1.0 / N) - mu * mu
    a = g_ref[...] * lax.rsqrt(var + 1e-5)
    bb = be_ref[...] - mu * a
    sumb = q0_ref[...] + q1_ref[...]
    degr = d0_ref[...] + d1_ref[...]
    deg = jnp.maximum(degr, 1.0)
    meanh = jnp.where(degr > 0.5, a * (sumb / deg) + bb, 0.0)
    h = a * hr_ref[...] + bb
    out = (lax.dot_general(meanh, wl_ref[...], _DN,
                           preferred_element_type=jnp.float32)
           + bl_ref[...]
           + lax.dot_general(h, wr_ref[...], _DN,
                             preferred_element_type=jnp.float32))
    nrm = jnp.sqrt(jnp.sum(out * out, axis=1, keepdims=True))
    out_ref[...] = out / jnp.maximum(nrm, 1e-12)


def _row_spec(nrow=BLK):
    return pl.BlockSpec((nrow, D), lambda i: (i, 0))


def _full_spec(shape):
    return pl.BlockSpec(shape, lambda i: tuple(0 for _ in shape))


def _tc1(sum2, d0, d1, x_p, w_l, b_l, w_r):
    return pl.pallas_call(
        _tc1_body,
        grid=(NBLK,),
        in_specs=[
            pl.BlockSpec((BLK, D), lambda i: (i, 0)),
            pl.BlockSpec((BLK, D), lambda i: (i + NBLK, 0)),
            pl.BlockSpec((BLK, 1), lambda i: (i, 0)),
            pl.BlockSpec((BLK, 1), lambda i: (i, 0)),
            pl.BlockSpec((BLK, D), lambda i: (i, 0)),
            _full_spec((D, D)),
            _full_spec((1, D)),
            _full_spec((D, D)),
        ],
        out_specs=[
            pl.BlockSpec((BLK, D), lambda i: (i, 0)),
            _full_spec((1, D)),
            _full_spec((1, D)),
        ],
        out_shape=[
            jax.ShapeDtypeStruct((NP, D), jnp.float32),
            jax.ShapeDtypeStruct((1, D), jnp.float32),
            jax.ShapeDtypeStruct((1, D), jnp.float32),
        ],
        scratch_shapes=[
            pltpu.VMEM((1, D), jnp.float32),
            pltpu.VMEM((1, D), jnp.float32),
        ],
    )(sum2, sum2, d0, d1, x_p, w_l, b_l, w_r)


def _tc2(sum2b, d0, d1, hr, s1, s2, g, be, w_l, b_l, w_r):
    return pl.pallas_call(
        _tc2_body,
        grid=(NBLK,),
        in_specs=[
            pl.BlockSpec((BLK, D), lambda i: (i, 0)),
            pl.BlockSpec((BLK, D), lambda i: (i + NBLK, 0)),
            pl.BlockSpec((BLK, 1), lambda i: (i, 0)),
            pl.BlockSpec((BLK, 1), lambda i: (i, 0)),
            pl.BlockSpec((BLK, D), lambda i: (i, 0)),
            _full_spec((1, D)),
            _full_spec((1, D)),
            _full_spec((1, D)),
            _full_spec((1, D)),
            _full_spec((D, D)),
            _full_spec((1, D)),
            _full_spec((D, D)),
        ],
        out_specs=pl.BlockSpec((BLK, D), lambda i: (i, 0)),
        out_shape=jax.ShapeDtypeStruct((N, D), jnp.float32),
    )(sum2b, sum2b, d0, d1, hr, s1, s2, g, be, w_l, b_l, w_r)


def kernel(x, edge_index, W1_l, b1_l, W1_r, gamma, beta, W2_l, b2_l, W2_r):
    src = edge_index[0]
    dst = edge_index[1]
    sum2, deg2 = _make_sc_kernel(True)(x, src, dst)
    d0 = deg2[0:NP].reshape(NP, 1)
    d1 = deg2[NP:].reshape(NP, 1)
    hr, s1, s2 = _tc1(sum2, d0, d1, x, W1_l, b1_l.reshape(1, D), W1_r)
    sum2b = _make_sc_kernel(False)(hr, src, dst)
    if isinstance(sum2b, (tuple, list)):
        (sum2b,) = sum2b
    out = _tc2(sum2b, d0, d1, hr, s1, s2, gamma.reshape(1, D),
               beta.reshape(1, D), W2_l, b2_l.reshape(1, D), W2_r)
    return out


# speedup vs baseline: 3.4830x; 1.0001x over previous
import functools

import jax
import jax.numpy as jnp
from jax import lax
from jax.experimental import pallas as pl
from jax.experimental.pallas import tpu as pltpu
from jax.experimental.pallas import tpu_sc as plsc

N = 10000
E = 320000
D = 128

NC = 2
NS = 16
NW = NC * NS
EPW = E // NW
CHUNK = 80
NCHUNK = EPW // CHUNK
NP = 10240
ROWS_PER_TILE = NP // NS


@functools.cache
def _make_sc_kernel(with_deg):
    NSLOT = 2
    LOOKAHEAD = 1
    out_type = [jax.ShapeDtypeStruct((NC * NP, D), jnp.float32)]
    scratch = [
        pltpu.VMEM_SHARED((NP, D), jnp.float32),
        pltpu.VMEM((EPW,), jnp.int32),
        pltpu.VMEM((EPW,), jnp.int32),
        [pltpu.VMEM((CHUNK, D), jnp.float32) for _ in range(NSLOT)],
        [pltpu.SemaphoreType.DMA for _ in range(NSLOT)],
        pltpu.VMEM((16, D), jnp.float32),
    ]
    if with_deg:
        out_type.append(jax.ShapeDtypeStruct((NC * NP,), jnp.float32))
        scratch += [
            pltpu.VMEM_SHARED((NP,), jnp.float32),
            pltpu.VMEM((EPW // 2 + 16,), jnp.float32),
            pltpu.VMEM((ROWS_PER_TILE,), jnp.float32),
        ]

    mesh = plsc.VectorSubcoreMesh(core_axis_name="c", subcore_axis_name="s",
                                  num_cores=NC, num_subcores=NS)

    def body(*refs):
        if with_deg:
            (x_hbm, src_hbm, dst_hbm, sum_hbm, deg_hbm,
             acc, src_all, dst_all, rows, sems, zbuf,
             dacc, ones_v, dzbuf) = refs
        else:
            (x_hbm, src_hbm, dst_hbm, sum_hbm,
             acc, src_all, dst_all, rows, sems, zbuf) = refs
        c = lax.axis_index("c")
        s = lax.axis_index("s")
        wid = c * NS + s

        pltpu.sync_copy(src_hbm.at[pl.ds(wid * EPW, EPW)], src_all)
        pltpu.sync_copy(dst_hbm.at[pl.ds(wid * EPW, EPW)], dst_all)

        for b in range(NSLOT):
            pltpu.async_copy(
                x_hbm.at[src_all.at[pl.ds(b * CHUNK, CHUNK)]],
                rows[b], sems[b])

        def zfill(i, carry):
            r = i // (D // 16)
            col = (i % (D // 16)) * 16
            zbuf[r, pl.ds(col, 16)] = jnp.zeros((16,), jnp.float32)
            return carry
        lax.fori_loop(0, 16 * (D // 16), zfill, 0)
        if with_deg:
            def ofill(i, carry):
                ones_v[pl.ds(i * 16, 16)] = jnp.ones((16,), jnp.float32)
                return carry
            lax.fori_loop(0, (EPW // 2 + 16) // 16, ofill, 0)

            def dzfill(i, carry):
                dzbuf[pl.ds(i * 16, 16)] = jnp.zeros((16,), jnp.float32)
                return carry
            lax.fori_loop(0, ROWS_PER_TILE // 16, dzfill, 0)

        row0 = s * ROWS_PER_TILE

        def zcopy(i, carry):
            pltpu.sync_copy(zbuf, acc.at[pl.ds(row0 + i * 16, 16)])
            return carry
        lax.fori_loop(0, ROWS_PER_TILE // 16, zcopy, 0)
        if with_deg:
            pltpu.sync_copy(dzbuf, dacc.at[pl.ds(row0, ROWS_PER_TILE)])
        plsc.subcore_barrier()

        if with_deg:
            half = EPW // 2
            for hh in range(2):
                pltpu.sync_copy(
                    ones_v.at[pl.ds(0, half)],
                    dacc.at[dst_all.at[pl.ds(hh * half, half)]],
                    add=True)

        dummy = x_hbm.at[pl.ds(0, CHUNK)]
        NGROUP = (NCHUNK + LOOKAHEAD + NSLOT - 1) // NSLOT

        def pipe_group(g, carry):
            for b in range(NSLOT):
                t = g * NSLOT + b

                @pl.when((t >= NSLOT) & (t < NCHUNK))
                def _fire():
                    pltpu.async_copy(
                        x_hbm.at[src_all.at[pl.ds(t * CHUNK, CHUNK)]],
                        rows[b], sems[b])

                cc = t - LOOKAHEAD
                jc = (b - LOOKAHEAD) % NSLOT

                @pl.when((cc >= 0) & (cc < NCHUNK))
                def _consume():
                    didx = dst_all.at[pl.ds(cc * CHUNK, CHUNK)]
                    pltpu.make_async_copy(dummy, rows[jc], sems[jc]).wait()
                    pltpu.sync_copy(rows[jc], acc.at[didx], add=True)
            return carry
        lax.fori_loop(0, NGROUP, pipe_group, 0)
        plsc.subcore_barrier()

        out0 = c * NP + s * ROWS_PER_TILE
        pltpu.sync_copy(acc.at[pl.ds(s * ROWS_PER_TILE, ROWS_PER_TILE)],
                        sum_hbm.at[pl.ds(out0, ROWS_PER_TILE)])
        if with_deg:
            pltpu.sync_copy(dacc.at[pl.ds(s * ROWS_PER_TILE, ROWS_PER_TILE)],
                            deg_hbm.at[pl.ds(out0, ROWS_PER_TILE)])

    return pl.kernel(body, out_type=tuple(out_type), mesh=mesh,
                     scratch_types=tuple(scratch))


BLK = 2048
NBLK = NP // BLK
_DN = (((1,), (1,)), ((), ()))


def _tc1_body(p0_ref, p1_ref, d0_ref, d1_ref, x_ref, wl_ref, bl_ref, wr_ref,
              hr_ref, s1_ref, s2_ref, a1, a2):
    i = pl.program_id(0)
    summed = p0_ref[...] + p1_ref[...]
    deg = jnp.maximum(d0_ref[...] + d1_ref[...], 1.0)
    mean = summed / deg
    out = (lax.dot_general(mean, wl_ref[...], _DN,
                           preferred_element_type=jnp.float32)
           + bl_ref[...]
           + lax.dot_general(x_ref[...], wr_ref[...], _DN,
                             preferred_element_type=jnp.float32))
    nrm = jnp.sqrt(jnp.sum(out * out, axis=1, keepdims=True))
    hr = jnp.maximum(out / jnp.maximum(nrm, 1e-12), 0.0)
    hr_ref[...] = hr
    rowid = i * BLK + lax.broadcasted_iota(jnp.int32, (BLK, 1), 0)
    hrm = jnp.where(rowid < N, hr, 0.0)
    bs1 = jnp.sum(hrm, axis=0, keepdims=True)
    bs2 = jnp.sum(hrm * hrm, axis=0, keepdims=True)

    @pl.when(i == 0)
    def _init():
        a1[...] = bs1
        a2[...] = bs2

    @pl.when(i > 0)
    def _accum():
        a1[...] += bs1
        a2[...] += bs2

    @pl.when(i == NBLK - 1)
    def _emit():
        s1_ref[...] = a1[...]
        s2_ref[...] = a2[...]


def _tc2_body(q0_ref, q1_ref, d0_ref, d1_ref, hr_ref, s1_ref, s2_ref,
              g_ref, be_ref, wl_ref, bl_ref, wr_ref, out_ref):
    mu = s1_ref[...] * (1.0 / N)
    var = s2_ref[...] * (1.0 / N) - mu * mu
    a = g_ref[...] * lax.rsqrt(var + 1e-5)
    bb = be_ref[...] - mu * a
    sumb = q0_ref[...] + q1_ref[...]
    degr = d0_ref[...] + d1_ref[...]
    deg = jnp.maximum(degr, 1.0)
    meanh = jnp.where(degr > 0.5, a * (sumb / deg) + bb, 0.0)
    h = a * hr_ref[...] + bb
    out = (lax.dot_general(meanh, wl_ref[...], _DN,
                           preferred_element_type=jnp.float32)
           + bl_ref[...]
           + lax.dot_general(h, wr_ref[...], _DN,
                             preferred_element_type=jnp.float32))
    nrm = jnp.sqrt(jnp.sum(out * out, axis=1, keepdims=True))
    out_ref[...] = out / jnp.maximum(nrm, 1e-12)


def _row_spec(nrow=BLK):
    return pl.BlockSpec((nrow, D), lambda i: (i, 0))


def _full_spec(shape):
    return pl.BlockSpec(shape, lambda i: tuple(0 for _ in shape))


def _tc1(sum2, d0, d1, x_p, w_l, b_l, w_r):
    return pl.pallas_call(
        _tc1_body,
        grid=(NBLK,),
        in_specs=[
            pl.BlockSpec((BLK, D), lambda i: (i, 0)),
            pl.BlockSpec((BLK, D), lambda i: (i + NBLK, 0)),
            pl.BlockSpec((BLK, 1), lambda i: (i, 0)),
            pl.BlockSpec((BLK, 1), lambda i: (i, 0)),
            pl.BlockSpec((BLK, D), lambda i: (i, 0)),
            _full_spec((D, D)),
            _full_spec((1, D)),
            _full_spec((D, D)),
        ],
        out_specs=[
            pl.BlockSpec((BLK, D), lambda i: (i, 0)),
            _full_spec((1, D)),
            _full_spec((1, D)),
        ],
        out_shape=[
            jax.ShapeDtypeStruct((NP, D), jnp.float32),
            jax.ShapeDtypeStruct((1, D), jnp.float32),
            jax.ShapeDtypeStruct((1, D), jnp.float32),
        ],
        scratch_shapes=[
            pltpu.VMEM((1, D), jnp.float32),
            pltpu.VMEM((1, D), jnp.float32),
        ],
    )(sum2, sum2, d0, d1, x_p, w_l, b_l, w_r)


def _tc2(sum2b, d0, d1, hr, s1, s2, g, be, w_l, b_l, w_r):
    return pl.pallas_call(
        _tc2_body,
        grid=(NBLK,),
        in_specs=[
            pl.BlockSpec((BLK, D), lambda i: (i, 0)),
            pl.BlockSpec((BLK, D), lambda i: (i + NBLK, 0)),
            pl.BlockSpec((BLK, 1), lambda i: (i, 0)),
            pl.BlockSpec((BLK, 1), lambda i: (i, 0)),
            pl.BlockSpec((BLK, D), lambda i: (i, 0)),
            _full_spec((1, D)),
            _full_spec((1, D)),
            _full_spec((1, D)),
            _full_spec((1, D)),
            _full_spec((D, D)),
            _full_spec((1, D)),
            _full_spec((D, D)),
        ],
        out_specs=pl.BlockSpec((BLK, D), lambda i: (i, 0)),
        out_shape=jax.ShapeDtypeStruct((N, D), jnp.float32),
    )(sum2b, sum2b, d0, d1, hr, s1, s2, g, be, w_l, b_l, w_r)


def kernel(x, edge_index, W1_l, b1_l, W1_r, gamma, beta, W2_l, b2_l, W2_r):
    src = edge_index[0]
    dst = edge_index[1]
    sum2, deg2 = _make_sc_kernel(True)(x, src, dst)
    d0 = deg2[0:NP].reshape(NP, 1)
    d1 = deg2[NP:].reshape(NP, 1)
    hr, s1, s2 = _tc1(sum2, d0, d1, x, W1_l, b1_l.reshape(1, D), W1_r)
    sum2b = _make_sc_kernel(False)(hr, src, dst)
    if isinstance(sum2b, (tuple, list)):
        (sum2b,) = sum2b
    out = _tc2(sum2b, d0, d1, hr, s1, s2, gamma.reshape(1, D),
               beta.reshape(1, D), W2_l, b2_l.reshape(1, D), W2_r)
    return out
